# initial kernel scaffold (unmeasured)
import jax
import jax.numpy as jnp
from jax import lax
from jax.experimental import pallas as pl
from jax.experimental.pallas import tpu as pltpu

N_DEV = 8


def kernel(x, w_mat, scale_x, scale_w):
    m, k_local = x.shape
    _, n = w_mat.shape
    ch = m // N_DEV

    def body(x_ref, w_ref, sx_ref, sw_ref, out_ref,
             send_buf, recv_buf, send_sem, recv_sem, credit_sem):
        my = lax.axis_index("i")
        left = (my - 1) % N_DEV
        right = (my + 1) % N_DEV

        barrier = pltpu.get_barrier_semaphore()
        for nbr in (left, right):
            pl.semaphore_signal(barrier, inc=1, device_id=(nbr,),
                                device_id_type=pl.DeviceIdType.MESH)
        pl.semaphore_wait(barrier, 2)

        out_ref[:, :] = jnp.dot(
            x_ref[:, :].astype(jnp.bfloat16),
            w_ref[:, :].astype(jnp.bfloat16),
            preferred_element_type=jnp.float32,
        )

        pl.semaphore_signal(credit_sem, inc=1, device_id=(left,),
                            device_id_type=pl.DeviceIdType.MESH)

        scale = sx_ref[0] * sw_ref[0]
        n_steps = 2 * (N_DEV - 1)

        for s in range(n_steps):
            if s < N_DEV - 1:
                send_c = (my - s) % N_DEV
                recv_c = (my - s - 1) % N_DEV
            else:
                h = s - (N_DEV - 1)
                send_c = (my + 1 - h) % N_DEV
                recv_c = (my - h) % N_DEV

            send_buf[:, :] = out_ref[pl.ds(send_c * ch, ch), :]
            pl.semaphore_wait(credit_sem, 1)
            rdma = pltpu.make_async_remote_copy(
                src_ref=send_buf,
                dst_ref=recv_buf,
                send_sem=send_sem,
                recv_sem=recv_sem,
                device_id=(right,),
                device_id_type=pl.DeviceIdType.MESH,
            )
            rdma.start()
            rdma.wait()

            if s < N_DEV - 1:
                out_ref[pl.ds(recv_c * ch, ch), :] += recv_buf[:, :]
            else:
                out_ref[pl.ds(recv_c * ch, ch), :] = recv_buf[:, :]

            if s < n_steps - 1:
                pl.semaphore_signal(credit_sem, inc=1, device_id=(left,),
                                    device_id_type=pl.DeviceIdType.MESH)

            if s == N_DEV - 2:
                rc = (my + 1) % N_DEV
                y = out_ref[pl.ds(rc * ch, ch), :] * scale
                out_ref[pl.ds(rc * ch, ch), :] = y * jax.nn.sigmoid(y)

    return pl.pallas_call(
        body,
        out_shape=jax.ShapeDtypeStruct((m, n), jnp.float32),
        in_specs=[
            pl.BlockSpec(memory_space=pltpu.VMEM),
            pl.BlockSpec(memory_space=pltpu.VMEM),
            pl.BlockSpec(memory_space=pltpu.SMEM),
            pl.BlockSpec(memory_space=pltpu.SMEM),
        ],
        out_specs=pl.BlockSpec(memory_space=pltpu.VMEM),
        scratch_shapes=[
            pltpu.VMEM((ch, n), jnp.float32),
            pltpu.VMEM((ch, n), jnp.float32),
            pltpu.SemaphoreType.DMA,
            pltpu.SemaphoreType.DMA,
            pltpu.SemaphoreType.REGULAR,
        ],
        compiler_params=pltpu.CompilerParams(collective_id=0),
    )(x, w_mat, scale_x, scale_w)


# baseline (device time: 733914 ns/iter reference)
import jax
import jax.numpy as jnp
from jax import lax
from jax.experimental import pallas as pl
from jax.experimental.pallas import tpu as pltpu

N_DEV = 8


def kernel(x, w_mat, scale_x, scale_w):
    m, k_local = x.shape
    _, n = w_mat.shape
    ch = m // N_DEV

    def body(x_ref, w_ref, sx_ref, sw_ref, out_ref,
             send_buf, recv_buf, send_sem, recv_sem, credit_sem):
        my = lax.axis_index("i")
        left = (my - 1) % N_DEV
        right = (my + 1) % N_DEV

        barrier = pltpu.get_barrier_semaphore()
        for nbr in (left, right):
            pl.semaphore_signal(barrier, inc=1, device_id=(nbr,),
                                device_id_type=pl.DeviceIdType.MESH)
        pl.semaphore_wait(barrier, 2)

        out_ref[:, :] = jnp.dot(
            x_ref[:, :].astype(jnp.bfloat16),
            w_ref[:, :].astype(jnp.bfloat16),
            preferred_element_type=jnp.float32,
        )

        pl.semaphore_signal(credit_sem, inc=1, device_id=(left,),
                            device_id_type=pl.DeviceIdType.MESH)

        scale = sx_ref[0] * sw_ref[0]
        n_steps = 2 * (N_DEV - 1)

        for s in range(n_steps):
            if s < N_DEV - 1:
                send_c = (my - s) % N_DEV
                recv_c = (my - s - 1) % N_DEV
            else:
                h = s - (N_DEV - 1)
                send_c = (my + 1 - h) % N_DEV
                recv_c = (my - h) % N_DEV

            send_buf[:, :] = out_ref[pl.ds(send_c * ch, ch), :]
            pl.semaphore_wait(credit_sem, 1)
            rdma = pltpu.make_async_remote_copy(
                src_ref=send_buf,
                dst_ref=recv_buf,
                send_sem=send_sem,
                recv_sem=recv_sem,
                device_id=(right,),
                device_id_type=pl.DeviceIdType.MESH,
            )
            rdma.start()
            rdma.wait()

            if s < N_DEV - 1:
                out_ref[pl.ds(recv_c * ch, ch), :] += recv_buf[:, :]
            else:
                out_ref[pl.ds(recv_c * ch, ch), :] = recv_buf[:, :]

            if s < n_steps - 1:
                pl.semaphore_signal(credit_sem, inc=1, device_id=(left,),
                                    device_id_type=pl.DeviceIdType.MESH)

            if s == N_DEV - 2:
                rc = (my + 1) % N_DEV
                y = out_ref[pl.ds(rc * ch, ch), :] * scale
                out_ref[pl.ds(rc * ch, ch), :] = y * jax.nn.sigmoid(y)

    return pl.pallas_call(
        body,
        out_shape=jax.ShapeDtypeStruct((m, n), jnp.float32),
        in_specs=[
            pl.BlockSpec(memory_space=pltpu.VMEM),
            pl.BlockSpec(memory_space=pltpu.VMEM),
            pl.BlockSpec(memory_space=pltpu.SMEM),
            pl.BlockSpec(memory_space=pltpu.SMEM),
        ],
        out_specs=pl.BlockSpec(memory_space=pltpu.VMEM),
        scratch_shapes=[
            pltpu.VMEM((ch, n), jnp.float32),
            pltpu.VMEM((ch, n), jnp.float32),
            pltpu.SemaphoreType.DMA,
            pltpu.SemaphoreType.DMA,
            pltpu.SemaphoreType.REGULAR,
        ],
        compiler_params=pltpu.CompilerParams(
            collective_id=0,
            vmem_limit_bytes=100 * 1024 * 1024,
        ),
    )(x, w_mat, scale_x, scale_w)


# device time: 304340 ns/iter; 2.4115x vs baseline; 2.4115x over previous
import jax
import jax.numpy as jnp
from jax import lax
from jax.experimental import pallas as pl
from jax.experimental.pallas import tpu as pltpu

N_DEV = 8
M = 4096
BM = 512
NBLK = M // BM
MX, MY, MZ = 1, 3, 4
SCHED = ((MX, MY, MZ), (MY, MZ, MX), (MZ, MX, MY))
COLS = ((0, 768), (768, 640), (1408, 640))


def kernel(x, w_mat, scale_x, scale_w):
    m, k_local = x.shape
    _, n = w_mat.shape

    def body(x_hbm, w_ref, sx_ref, sw_ref, out_ref,
             xbuf, wcast, rbuf0, rbuf1, rbuf2,
             xsems, send_sems, recv_sems, credit_sems, exit_sem):
        my = lax.axis_index("i")
        b0 = my % 2
        b1 = (my // 2) % 2
        b2 = my // 4
        bit_of = {MX: (b0 + b1) % 2, MY: b1, MZ: b2}
        rbufs = [rbuf0, rbuf1, rbuf2]

        barrier = pltpu.get_barrier_semaphore()
        for mk in (MX, MY, MZ):
            pl.semaphore_signal(barrier, inc=1, device_id=(my ^ mk,),
                                device_id_type=pl.DeviceIdType.MESH)
        pl.semaphore_wait(barrier, 3)

        wcast[:, :] = w_ref[:, :].astype(jnp.bfloat16)
        copies = []
        c0 = pltpu.make_async_copy(
            x_hbm.at[pl.ds(0, BM), :], xbuf.at[0], xsems.at[0])
        c0.start()
        copies.append(c0)
        for mb in range(NBLK):
            if mb + 1 < NBLK:
                nxt = pltpu.make_async_copy(
                    x_hbm.at[pl.ds((mb + 1) * BM, BM), :],
                    xbuf.at[(mb + 1) % 3], xsems.at[(mb + 1) % 3])
                nxt.start()
                copies.append(nxt)
            copies[mb].wait()
            out_ref[pl.ds(mb * BM, BM), :] = jnp.dot(
                xbuf[mb % 3].astype(jnp.bfloat16),
                wcast[:, :],
                preferred_element_type=jnp.float32,
            )

        seg_start = [0, 0, 0]
        for p in range(3):
            half = M >> (p + 1)
            started = []
            for j in range(3):
                mk = SCHED[j][p]
                b = bit_of[mk]
                kept = seg_start[j] + b * half
                sent = seg_start[j] + (1 - b) * half
                cs, cw = COLS[j]
                if p > 0:
                    pl.semaphore_wait(credit_sems.at[j], 1)
                rdma = pltpu.make_async_remote_copy(
                    src_ref=out_ref.at[pl.ds(sent, half), pl.ds(cs, cw)],
                    dst_ref=rbufs[j].at[pl.ds(0, half), :],
                    send_sem=send_sems.at[j],
                    recv_sem=recv_sems.at[j, p],
                    device_id=(my ^ mk,),
                    device_id_type=pl.DeviceIdType.MESH,
                )
                rdma.start()
                started.append((rdma, j, kept))
                seg_start[j] = kept
            for rdma, j, kept in started:
                rdma.wait()
                cs, cw = COLS[j]
                out_ref[pl.ds(kept, half), pl.ds(cs, cw)] += (
                    rbufs[j][pl.ds(0, half), :]
                )
                if p < 2:
                    pl.semaphore_signal(
                        credit_sems.at[j], inc=1,
                        device_id=(my ^ SCHED[j][p + 1],),
                        device_id_type=pl.DeviceIdType.MESH)

        scale = sx_ref[0] * sw_ref[0]
        seg = M // N_DEV
        for j in range(3):
            cs, cw = COLS[j]
            y = out_ref[pl.ds(seg_start[j], seg), pl.ds(cs, cw)] * scale
            out_ref[pl.ds(seg_start[j], seg), pl.ds(cs, cw)] = (
                y * jax.nn.sigmoid(y)
            )

        for i, p in enumerate((2, 1, 0)):
            size = M >> (p + 1)
            started = []
            for j in range(3):
                mk = SCHED[j][p]
                b = bit_of[mk]
                r = seg_start[j]
                cs, cw = COLS[j]
                rdma = pltpu.make_async_remote_copy(
                    src_ref=out_ref.at[pl.ds(r, size), pl.ds(cs, cw)],
                    dst_ref=out_ref.at[pl.ds(r, size), pl.ds(cs, cw)],
                    send_sem=send_sems.at[j],
                    recv_sem=recv_sems.at[j, 3 + i],
                    device_id=(my ^ mk,),
                    device_id_type=pl.DeviceIdType.MESH,
                )
                rdma.start()
                started.append(rdma)
                seg_start[j] = r - b * size
            for rdma in started:
                rdma.wait()

        for mk in (MX, MY, MZ):
            pl.semaphore_signal(exit_sem, inc=1, device_id=(my ^ mk,),
                                device_id_type=pl.DeviceIdType.MESH)
        pl.semaphore_wait(exit_sem, 3)

    return pl.pallas_call(
        body,
        out_shape=jax.ShapeDtypeStruct((m, n), jnp.float32),
        in_specs=[
            pl.BlockSpec(memory_space=pltpu.MemorySpace.HBM),
            pl.BlockSpec(memory_space=pltpu.VMEM),
            pl.BlockSpec(memory_space=pltpu.SMEM),
            pl.BlockSpec(memory_space=pltpu.SMEM),
        ],
        out_specs=pl.BlockSpec(memory_space=pltpu.VMEM),
        scratch_shapes=[
            pltpu.VMEM((3, BM, k_local), jnp.float32),
            pltpu.VMEM((k_local, n), jnp.bfloat16),
            pltpu.VMEM((M // 2, COLS[0][1]), jnp.float32),
            pltpu.VMEM((M // 2, COLS[1][1]), jnp.float32),
            pltpu.VMEM((M // 2, COLS[2][1]), jnp.float32),
            pltpu.SemaphoreType.DMA((3,)),
            pltpu.SemaphoreType.DMA((3,)),
            pltpu.SemaphoreType.DMA((3, 6)),
            pltpu.SemaphoreType.REGULAR((3,)),
            pltpu.SemaphoreType.REGULAR,
        ],
        compiler_params=pltpu.CompilerParams(
            collective_id=0,
            vmem_limit_bytes=63 * 1024 * 1024,
        ),
    )(x, w_mat, scale_x, scale_w)
